# baseline (speedup 1.0000x reference)
import functools

import jax
import jax.numpy as jnp
from jax import lax
from jax.experimental import pallas as pl
from jax.experimental.pallas import tpu as pltpu
from jax.experimental.pallas import tpu_sc as plsc

N = 10000
DIN = 128
DH = 512
NCHUNK = 4
CW = 144
E_TRUE = 330000
E_PAD = 330240
EPS_BN = 1e-5

PA_B = 48
PA_PER_TILE = E_PAD // 32
PA_NBLK = PA_PER_TILE // PA_B

PC_B = 80
PC_PER_TILE = E_PAD // 16
PC_NBLK = PC_PER_TILE // PC_B
PC_DEPTH = 3
ROWS_PER_TILE = N // 16
STRIP = 25
NSTRIP = ROWS_PER_TILE // STRIP

ROW_BLK = 1000
NROW_BLK = N // ROW_BLK


def _sc_mesh():
    return plsc.VectorSubcoreMesh(core_axis_name="c", subcore_axis_name="s")


def _build_pass_a():
    @functools.partial(
        pl.kernel,
        mesh=_sc_mesh(),
        compiler_params=pltpu.CompilerParams(
            use_tc_tiling_on_sc=False, needs_layout_passes=False),
        out_type=jax.ShapeDtypeStruct((E_PAD,), jnp.float32),
        scratch_types=[
            pltpu.VMEM((PA_B,), jnp.int32),
            pltpu.VMEM((PA_B,), jnp.int32),
            pltpu.VMEM((PA_B,), jnp.int32),
            pltpu.VMEM((PA_B,), jnp.int32),
            pltpu.VMEM((PA_B, DH), jnp.float32),
            pltpu.VMEM((PA_B, DH), jnp.float32),
            pltpu.VMEM((PA_B, DH), jnp.float32),
            pltpu.VMEM((PA_B, DH), jnp.float32),
            pltpu.VMEM((DH,), jnp.float32),
            pltpu.VMEM((PA_B,), jnp.float32),
            pltpu.VMEM((PA_B,), jnp.float32),
            pltpu.SemaphoreType.DMA,
            pltpu.SemaphoreType.DMA,
        ],
    )
    def pass_a(src_hbm, dst_hbm, xl_hbm, xr_hbm, att_hbm, ex_hbm,
               idxl0, idxl1, idxr0, idxr1, rl0, rl1, rr0, rr1,
               att_v, exb0, exb1, sem0, sem1):
        cc = lax.axis_index("c")
        ss = lax.axis_index("s")
        wid = ss * 2 + cc
        base = wid * PA_PER_TILE
        pltpu.sync_copy(att_hbm, att_v)
        attg = [att_v[pl.ds(d * 16, 16)] for d in range(DH // 16)]

        lanes = lax.iota(jnp.int32, 16)
        idxl = [idxl0, idxl1]
        idxr = [idxr0, idxr1]
        rl = [rl0, rl1]
        rr = [rr0, rr1]
        exb = [exb0, exb1]
        sems = [sem0, sem1]

        def load(k, p):
            off = base + k * PA_B
            pltpu.sync_copy(src_hbm.at[pl.ds(off, PA_B)], idxl[p])
            pltpu.sync_copy(dst_hbm.at[pl.ds(off, PA_B)], idxr[p])
            pltpu.async_copy(xl_hbm.at[idxl[p]], rl[p], sems[p])
            pltpu.async_copy(xr_hbm.at[idxr[p]], rr[p], sems[p])

        def compute(k, p):
            pltpu.make_async_copy(xl_hbm.at[idxl[p]], rl[p], sems[p]).wait()
            pltpu.make_async_copy(xr_hbm.at[idxr[p]], rr[p], sems[p]).wait()
            off = base + k * PA_B
            rlp, rrp, exp_ = rl[p], rr[p], exb[p]

            def group(g, c2):
                def edge(eu, svec):
                    e = g * 16 + eu
                    acc = [jnp.zeros((16,), jnp.float32) for _ in range(4)]
                    for d in range(DH // 16):
                        zl = rlp[e, pl.ds(d * 16, 16)]
                        zr = rrp[e, pl.ds(d * 16, 16)]
                        z = zl + zr
                        lr = jnp.maximum(z, 0.2 * z)
                        acc[d % 4] = acc[d % 4] + lr * attg[d]
                    s = jnp.sum((acc[0] + acc[1]) + (acc[2] + acc[3]))
                    return jnp.where(lanes == eu, s, svec)

                svec = lax.fori_loop(0, 16, edge,
                                     jnp.zeros((16,), jnp.float32))
                gid = off + g * 16 + lanes
                ev = jnp.where(gid < E_TRUE, jnp.exp(svec), 0.0)
                exp_[pl.ds(g * 16, 16)] = ev
                return c2

            lax.fori_loop(0, PA_B // 16, group, 0)
            pltpu.sync_copy(exp_, ex_hbm.at[pl.ds(off, PA_B)])

        load(0, 0)

        def pair(kk, carry):
            k0 = kk * 2
            load(k0 + 1, 1)
            compute(k0, 0)
            load(k0 + 2, 0)
            compute(k0 + 1, 1)
            return carry

        lax.fori_loop(0, (PA_NBLK - 1) // 2, pair, 0)
        compute(PA_NBLK - 1, (PA_NBLK - 1) % 2)

    return pass_a


def _build_pass_c():
    @functools.partial(
        pl.kernel,
        mesh=_sc_mesh(),
        compiler_params=pltpu.CompilerParams(
            use_tc_tiling_on_sc=False, needs_layout_passes=False),
        out_type=jax.ShapeDtypeStruct((NCHUNK * N, CW), jnp.float32),
        scratch_types=[
            pltpu.VMEM_SHARED((N, CW), jnp.float32),
            pltpu.VMEM((PC_B,), jnp.int32),
            pltpu.VMEM((PC_B,), jnp.int32),
            pltpu.VMEM((PC_B,), jnp.int32),
            pltpu.VMEM((PC_B,), jnp.int32),
            pltpu.VMEM((PC_B,), jnp.int32),
            pltpu.VMEM((PC_B,), jnp.int32),
            pltpu.VMEM((PC_B,), jnp.float32),
            pltpu.VMEM((PC_B,), jnp.float32),
            pltpu.VMEM((PC_B,), jnp.float32),
            pltpu.VMEM((PC_B, CW), jnp.float32),
            pltpu.VMEM((PC_B, CW), jnp.float32),
            pltpu.VMEM((PC_B, CW), jnp.float32),
            pltpu.VMEM((STRIP, CW), jnp.float32),
            pltpu.SemaphoreType.DMA,
            pltpu.SemaphoreType.DMA,
            pltpu.SemaphoreType.DMA,
            pltpu.SemaphoreType.DMA,
            pltpu.SemaphoreType.DMA,
            pltpu.SemaphoreType.DMA,
        ],
    )
    def pass_c(src_hbm, dst_hbm, ex_hbm, tab_hbm, out_hbm,
               acc_sh, idxd0, idxd1, idxd2, gidx0, gidx1, gidx2,
               exv0, exv1, exv2, rows0, rows1, rows2, strip_v,
               semg0, semg1, semg2, sems0, sems1, sems2):
        cc = lax.axis_index("c")
        ss = lax.axis_index("s")
        ebase = ss * PC_PER_TILE

        z16 = jnp.zeros((16,), jnp.float32)
        idxd = [idxd0, idxd1, idxd2]
        gidx = [gidx0, gidx1, gidx2]
        exv = [exv0, exv1, exv2]
        rows = [rows0, rows1, rows2]
        semg = [semg0, semg1, semg2]
        sems = [sems0, sems1, sems2]

        for j in range(2):
            chunk = cc * 2 + j

            def zz(i, c2):
                r = i // (CW // 16)
                g = i % (CW // 16)
                strip_v[r, pl.ds(g * 16, 16)] = z16
                return c2

            lax.fori_loop(0, STRIP * (CW // 16), zz, 0)
            for t in range(NSTRIP):
                pltpu.sync_copy(
                    strip_v,
                    acc_sh.at[pl.ds(ss * ROWS_PER_TILE + t * STRIP, STRIP)])
            plsc.subcore_barrier()

            def load(k, p, wait_pred):
                if wait_pred is not None:
                    @pl.when(wait_pred)
                    def _():
                        pltpu.make_async_copy(
                            rows[p], acc_sh.at[idxd[p]], sems[p]).wait()
                off = ebase + k * PC_B
                pltpu.sync_copy(dst_hbm.at[pl.ds(off, PC_B)], idxd[p])
                pltpu.sync_copy(src_hbm.at[pl.ds(off, PC_B)], gidx[p])
                pltpu.sync_copy(ex_hbm.at[pl.ds(off, PC_B)], exv[p])

                def mkidx(i, c2):
                    gidx[p][pl.ds(i * 16, 16)] = (
                        gidx[p][pl.ds(i * 16, 16)] + chunk * N)
                    return c2

                lax.fori_loop(0, PC_B // 16, mkidx, 0)
                pltpu.async_copy(tab_hbm.at[gidx[p]], rows[p], semg[p])

            def proc(k, p):
                pltpu.make_async_copy(
                    tab_hbm.at[gidx[p]], rows[p], semg[p]).wait()
                rp, ep = rows[p], exv[p]

                def scale(eg, c2):
                    ev = ep[pl.ds(eg * 16, 16)]
                    for eu in range(16):
                        e = eg * 16 + eu
                        w = ev[eu]
                        for g in range(CW // 16):
                            rp[e, pl.ds(g * 16, 16)] = (
                                rp[e, pl.ds(g * 16, 16)] * w)
                    return c2

                lax.fori_loop(0, PC_B // 16, scale, 0)
                pltpu.async_copy(rp, acc_sh.at[idxd[p]], sems[p], add=True)

            load(0, 0, None)
            load(1, 1, None)

            def rot(kk, carry):
                for r in range(PC_DEPTH):
                    k = kk * PC_DEPTH + r
                    m = k + 2

                    @pl.when(m < PC_NBLK)
                    def _():
                        load(m, (r + 2) % PC_DEPTH, m >= PC_DEPTH)

                    proc(k, r)
                return carry

            lax.fori_loop(0, PC_NBLK // PC_DEPTH, rot, 0)
            for p in range(PC_DEPTH):
                pltpu.make_async_copy(
                    rows[p], acc_sh.at[idxd[p]], sems[p]).wait()

            plsc.subcore_barrier()
            for t in range(NSTRIP):
                r0 = ss * ROWS_PER_TILE + t * STRIP
                pltpu.sync_copy(acc_sh.at[pl.ds(r0, STRIP)], strip_v)
                pltpu.sync_copy(strip_v, out_hbm.at[pl.ds(chunk * N + r0, STRIP)])
            plsc.subcore_barrier()

    return pass_c


def _dense_prologue(x, wl, bl, wr, br):
    def body(x_ref, wl_ref, bl_ref, wr_ref, br_ref, xl_ref, xr_ref):
        xb = x_ref[...]
        dn = (((1,), (1,)), ((), ()))
        xl_ref[...] = lax.dot_general(
            xb, wl_ref[...], dn, preferred_element_type=jnp.float32) + bl_ref[...]
        xr_ref[...] = lax.dot_general(
            xb, wr_ref[...], dn, preferred_element_type=jnp.float32) + br_ref[...]

    return pl.pallas_call(
        body,
        grid=(NROW_BLK,),
        in_specs=[
            pl.BlockSpec((ROW_BLK, DIN), lambda j: (j, 0)),
            pl.BlockSpec((DH, DIN), lambda j: (0, 0)),
            pl.BlockSpec((1, DH), lambda j: (0, 0)),
            pl.BlockSpec((DH, DIN), lambda j: (0, 0)),
            pl.BlockSpec((1, DH), lambda j: (0, 0)),
        ],
        out_specs=[
            pl.BlockSpec((ROW_BLK, DH), lambda j: (j, 0)),
            pl.BlockSpec((ROW_BLK, DH), lambda j: (j, 0)),
        ],
        out_shape=[
            jax.ShapeDtypeStruct((N, DH), jnp.float32),
            jax.ShapeDtypeStruct((N, DH), jnp.float32),
        ],
    )(x, wl, bl.reshape(1, DH), wr, br.reshape(1, DH))


def _gat_finish(num, den, bias):
    def body(num_ref, den_ref, bias_ref, a0_ref, st_ref, acc):
        j = pl.program_id(0)
        a = num_ref[...] / (den_ref[...] + 1e-16) + bias_ref[...]
        a = jnp.maximum(a, 0.0)
        a0_ref[...] = a

        @pl.when(j == 0)
        def _():
            acc[...] = jnp.zeros_like(acc)

        acc[0:1, :] = acc[0:1, :] + jnp.sum(a, axis=0, keepdims=True)
        acc[1:2, :] = acc[1:2, :] + jnp.sum(a * a, axis=0, keepdims=True)

        @pl.when(j == pl.num_programs(0) - 1)
        def _():
            st_ref[...] = acc[...]

    return pl.pallas_call(
        body,
        grid=(NROW_BLK,),
        in_specs=[
            pl.BlockSpec((ROW_BLK, DH), lambda j: (j, 0)),
            pl.BlockSpec((ROW_BLK, 1), lambda j: (j, 0)),
            pl.BlockSpec((1, DH), lambda j: (0, 0)),
        ],
        out_specs=[
            pl.BlockSpec((ROW_BLK, DH), lambda j: (j, 0)),
            pl.BlockSpec((8, DH), lambda j: (0, 0)),
        ],
        out_shape=[
            jax.ShapeDtypeStruct((N, DH), jnp.float32),
            jax.ShapeDtypeStruct((8, DH), jnp.float32),
        ],
        scratch_shapes=[pltpu.VMEM((8, DH), jnp.float32)],
    )(num, den, bias.reshape(1, DH))


def _bn_matmul_relu(a, st, g, be, w, b, d_in, d_out):
    def body(a_ref, st_ref, g_ref, be_ref, w_ref, b_ref, y_ref, stout_ref, acc):
        j = pl.program_id(0)
        m = st_ref[0:1, :] / N
        v = st_ref[1:2, :] / N - m * m
        inv = lax.rsqrt(v + EPS_BN)
        xn = (a_ref[...] - m) * inv * g_ref[...] + be_ref[...]
        dn = (((1,), (1,)), ((), ()))
        y = lax.dot_general(
            xn, w_ref[...], dn, preferred_element_type=jnp.float32) + b_ref[...]
        y = jnp.maximum(y, 0.0)
        y_ref[...] = y

        @pl.when(j == 0)
        def _():
            acc[...] = jnp.zeros_like(acc)

        acc[0:1, :] = acc[0:1, :] + jnp.sum(y, axis=0, keepdims=True)
        acc[1:2, :] = acc[1:2, :] + jnp.sum(y * y, axis=0, keepdims=True)

        @pl.when(j == pl.num_programs(0) - 1)
        def _():
            stout_ref[...] = acc[...]

    return pl.pallas_call(
        body,
        grid=(NROW_BLK,),
        in_specs=[
            pl.BlockSpec((ROW_BLK, d_in), lambda j: (j, 0)),
            pl.BlockSpec((8, d_in), lambda j: (0, 0)),
            pl.BlockSpec((1, d_in), lambda j: (0, 0)),
            pl.BlockSpec((1, d_in), lambda j: (0, 0)),
            pl.BlockSpec((d_out, d_in), lambda j: (0, 0)),
            pl.BlockSpec((1, d_out), lambda j: (0, 0)),
        ],
        out_specs=[
            pl.BlockSpec((ROW_BLK, d_out), lambda j: (j, 0)),
            pl.BlockSpec((8, d_out), lambda j: (0, 0)),
        ],
        out_shape=[
            jax.ShapeDtypeStruct((N, d_out), jnp.float32),
            jax.ShapeDtypeStruct((8, d_out), jnp.float32),
        ],
        scratch_shapes=[pltpu.VMEM((8, d_out), jnp.float32)],
    )(a, st, g.reshape(1, d_in), be.reshape(1, d_in), w, b.reshape(1, d_out))


def _head(a, st, g, be, w, b, d_in, d_out):
    def body(a_ref, st_ref, g_ref, be_ref, w_ref, b_ref, sig_ref, lsm_ref):
        m = st_ref[0:1, :] / N
        v = st_ref[1:2, :] / N - m * m
        inv = lax.rsqrt(v + EPS_BN)
        xn = (a_ref[...] - m) * inv * g_ref[...] + be_ref[...]
        dn = (((1,), (1,)), ((), ()))
        z = lax.dot_general(
            xn, w_ref[...], dn, preferred_element_type=jnp.float32) + b_ref[...]
        sig_ref[...] = 1.0 / (1.0 + jnp.exp(-z))
        zm = jnp.max(z, axis=1, keepdims=True)
        lse = zm + jnp.log(jnp.sum(jnp.exp(z - zm), axis=1, keepdims=True))
        lsm_ref[...] = z - lse

    return pl.pallas_call(
        body,
        grid=(NROW_BLK,),
        in_specs=[
            pl.BlockSpec((ROW_BLK, d_in), lambda j: (j, 0)),
            pl.BlockSpec((8, d_in), lambda j: (0, 0)),
            pl.BlockSpec((1, d_in), lambda j: (0, 0)),
            pl.BlockSpec((1, d_in), lambda j: (0, 0)),
            pl.BlockSpec((d_out, d_in), lambda j: (0, 0)),
            pl.BlockSpec((1, d_out), lambda j: (0, 0)),
        ],
        out_specs=[
            pl.BlockSpec((ROW_BLK, d_out), lambda j: (j, 0)),
            pl.BlockSpec((ROW_BLK, d_out), lambda j: (j, 0)),
        ],
        out_shape=[
            jax.ShapeDtypeStruct((N, d_out), jnp.float32),
            jax.ShapeDtypeStruct((N, d_out), jnp.float32),
        ],
    )(a, st, g.reshape(1, d_in), be.reshape(1, d_in), w, b.reshape(1, d_out))


def kernel(data_0, data_1, edge_index_0, edge_index_1, batch_0, batch_1,
           train, params):
    x = data_1
    sl = jnp.arange(N, dtype=jnp.int32)
    pad = jnp.zeros((E_PAD - E_TRUE,), jnp.int32)
    src = jnp.concatenate([edge_index_1[0], sl, pad])
    dst = jnp.concatenate([edge_index_1[1], sl, pad])

    xl, xr = _dense_prologue(x, params['Wl1'], params['bl1'],
                             params['Wr1'], params['br1'])

    xl4 = xl.reshape(N, NCHUNK, DIN).transpose(1, 0, 2)
    ones = jnp.ones((NCHUNK, N, 1), jnp.float32)
    zpad = jnp.zeros((NCHUNK, N, CW - DIN - 1), jnp.float32)
    tab = jnp.concatenate([xl4, ones, zpad], axis=2).reshape(NCHUNK * N, CW)

    ex = _build_pass_a()(src, dst, xl, xr, params['att1'])
    acc = _build_pass_c()(src, dst, ex, tab)

    acc4 = acc.reshape(NCHUNK, N, CW)
    num = acc4[:, :, :DIN].transpose(1, 0, 2).reshape(N, DH)
    den = acc4[0, :, DIN:DIN + 1]

    a0, st0 = _gat_finish(num, den, params['bias1'])
    a1, st1 = _bn_matmul_relu(a0, st0, params['bng1'], params['bnb1'],
                              params['W1'], params['b1'], DH, 256)
    a2, st2 = _bn_matmul_relu(a1, st1, params['g1'], params['be1'],
                              params['W2'], params['b2'], 256, 128)
    sig, lsm = _head(a2, st2, params['g2'], params['be2'],
                     params['W3'], params['b3'], 128, 8)
    return (sig, lsm)


# speedup vs baseline: 1.2817x; 1.2817x over previous
import functools

import jax
import jax.numpy as jnp
from jax import lax
from jax.experimental import pallas as pl
from jax.experimental.pallas import tpu as pltpu
from jax.experimental.pallas import tpu_sc as plsc

N = 10000
DIN = 128
DH = 512
NCHUNK = 4
CW = 144
E_TRUE = 330000
E_PAD = 330240
EPS_BN = 1e-5

PA_B = 48
PA_PER_TILE = E_PAD // 32
PA_NBLK = PA_PER_TILE // PA_B

PC_B = 96
PC_PER_TILE = E_PAD // 16
PC_NBLK = PC_PER_TILE // PC_B
ROWS_PER_TILE = N // 16
STRIP = 25
NSTRIP = ROWS_PER_TILE // STRIP

ROW_BLK = 1000
NROW_BLK = N // ROW_BLK


def _sc_mesh():
    return plsc.VectorSubcoreMesh(core_axis_name="c", subcore_axis_name="s")


def _build_pass_a():
    @functools.partial(
        pl.kernel,
        mesh=_sc_mesh(),
        compiler_params=pltpu.CompilerParams(
            use_tc_tiling_on_sc=False, needs_layout_passes=False),
        out_type=jax.ShapeDtypeStruct((E_PAD,), jnp.float32),
        scratch_types=[
            pltpu.VMEM((2, PA_B), jnp.int32),
            pltpu.VMEM((2, PA_B), jnp.int32),
            pltpu.VMEM((PA_B,), jnp.int32),
            pltpu.VMEM((PA_B,), jnp.int32),
            pltpu.VMEM((PA_B,), jnp.int32),
            pltpu.VMEM((PA_B,), jnp.int32),
            pltpu.VMEM((PA_B, DH), jnp.float32),
            pltpu.VMEM((PA_B, DH), jnp.float32),
            pltpu.VMEM((PA_B, DH), jnp.float32),
            pltpu.VMEM((PA_B, DH), jnp.float32),
            pltpu.VMEM((DH,), jnp.float32),
            pltpu.VMEM((PA_B,), jnp.float32),
            pltpu.VMEM((PA_B,), jnp.float32),
            pltpu.SemaphoreType.DMA,
            pltpu.SemaphoreType.DMA,
            pltpu.SemaphoreType.DMA,
            pltpu.SemaphoreType.DMA,
        ],
    )
    def pass_a(ed_hbm, xl_hbm, xr_hbm, att_hbm, ex_hbm,
               eb0, eb1, idxl0, idxl1, idxr0, idxr1, rl0, rl1, rr0, rr1,
               att_v, exb0, exb1, sem0, sem1, seme0, seme1):
        cc = lax.axis_index("c")
        ss = lax.axis_index("s")
        wid = ss * 2 + cc
        bbase = wid * PA_NBLK
        pltpu.sync_copy(att_hbm, att_v)
        attg = [att_v[pl.ds(d * 16, 16)] for d in range(DH // 16)]

        lanes = lax.iota(jnp.int32, 16)
        eb = [eb0, eb1]
        idxl = [idxl0, idxl1]
        idxr = [idxr0, idxr1]
        rl = [rl0, rl1]
        rr = [rr0, rr1]
        exb = [exb0, exb1]
        sems = [sem0, sem1]
        seme = [seme0, seme1]

        def load(k, p):
            pltpu.sync_copy(ed_hbm.at[bbase + k], eb[p])
            for i in range(PA_B // 16):
                idxl[p][pl.ds(i * 16, 16)] = eb[p][0, pl.ds(i * 16, 16)]
                idxr[p][pl.ds(i * 16, 16)] = eb[p][1, pl.ds(i * 16, 16)]
            pltpu.async_copy(xl_hbm.at[idxl[p]], rl[p], sems[p])
            pltpu.async_copy(xr_hbm.at[idxr[p]], rr[p], sems[p])

        def compute(k, p, wait_ex):
            pltpu.make_async_copy(xl_hbm.at[idxl[p]], rl[p], sems[p]).wait()
            pltpu.make_async_copy(xr_hbm.at[idxr[p]], rr[p], sems[p]).wait()
            off = (bbase + k) * PA_B
            rlp, rrp, exp_ = rl[p], rr[p], exb[p]
            if wait_ex is not None:
                @pl.when(wait_ex)
                def _():
                    pltpu.make_async_copy(
                        exp_, ex_hbm.at[pl.ds(off, PA_B)], seme[p]).wait()

            def group(g, c2):
                def edge(eu, svec):
                    e = g * 16 + eu
                    acc = [jnp.zeros((16,), jnp.float32) for _ in range(4)]
                    for d in range(DH // 16):
                        zl = rlp[e, pl.ds(d * 16, 16)]
                        zr = rrp[e, pl.ds(d * 16, 16)]
                        z = zl + zr
                        lr = jnp.maximum(z, 0.2 * z)
                        acc[d % 4] = acc[d % 4] + lr * attg[d]
                    s = jnp.sum((acc[0] + acc[1]) + (acc[2] + acc[3]))
                    return jnp.where(lanes == eu, s, svec)

                svec = lax.fori_loop(0, 16, edge,
                                     jnp.zeros((16,), jnp.float32))
                gid = off + g * 16 + lanes
                ev = jnp.where(gid < E_TRUE, jnp.exp(svec), 0.0)
                exp_[pl.ds(g * 16, 16)] = ev
                return c2

            lax.fori_loop(0, PA_B // 16, group, 0)
            pltpu.async_copy(exp_, ex_hbm.at[pl.ds(off, PA_B)], seme[p])

        load(0, 0)

        def pair(kk, carry):
            k0 = kk * 2
            load(k0 + 1, 1)
            compute(k0, 0, kk >= 1)
            load(k0 + 2, 0)
            compute(k0 + 1, 1, kk >= 1)
            return carry

        lax.fori_loop(0, (PA_NBLK - 1) // 2, pair, 0)
        compute(PA_NBLK - 1, (PA_NBLK - 1) % 2, True)
        kl = PA_NBLK - 1
        pltpu.make_async_copy(
            exb[kl % 2], ex_hbm.at[pl.ds((bbase + kl) * PA_B, PA_B)],
            seme[kl % 2]).wait()
        kl2 = PA_NBLK - 2
        pltpu.make_async_copy(
            exb[kl2 % 2], ex_hbm.at[pl.ds((bbase + kl2) * PA_B, PA_B)],
            seme[kl2 % 2]).wait()

    return pass_a


def _build_pass_c():
    @functools.partial(
        pl.kernel,
        mesh=_sc_mesh(),
        compiler_params=pltpu.CompilerParams(
            use_tc_tiling_on_sc=False, needs_layout_passes=False),
        out_type=jax.ShapeDtypeStruct((NCHUNK * N, CW), jnp.float32),
        scratch_types=[
            pltpu.VMEM_SHARED((N, CW), jnp.float32),
            pltpu.VMEM((3, PC_B), jnp.int32),
            pltpu.VMEM((3, PC_B), jnp.int32),
            pltpu.VMEM((PC_B,), jnp.int32),
            pltpu.VMEM((PC_B,), jnp.int32),
            pltpu.VMEM((PC_B,), jnp.int32),
            pltpu.VMEM((PC_B,), jnp.int32),
            pltpu.VMEM((PC_B,), jnp.float32),
            pltpu.VMEM((PC_B,), jnp.float32),
            pltpu.VMEM((PC_B, CW), jnp.float32),
            pltpu.VMEM((PC_B, CW), jnp.float32),
            pltpu.VMEM((STRIP, CW), jnp.float32),
            pltpu.SemaphoreType.DMA,
            pltpu.SemaphoreType.DMA,
        ],
    )
    def pass_c(ed_hbm, tab_hbm, out_hbm,
               acc_sh, eb0, eb1, idxd0, idxd1, gidx0, gidx1,
               exv0, exv1, rows0, rows1, strip_v, sem0, sem1):
        cc = lax.axis_index("c")
        ss = lax.axis_index("s")
        bbase = ss * PC_NBLK

        z16 = jnp.zeros((16,), jnp.float32)
        eb = [eb0, eb1]
        idxd = [idxd0, idxd1]
        gidx = [gidx0, gidx1]
        exv = [exv0, exv1]
        rows = [rows0, rows1]
        sems = [sem0, sem1]

        for j in range(2):
            chunk = cc * 2 + j

            def zz(i, c2):
                r = i // (CW // 16)
                g = i % (CW // 16)
                strip_v[r, pl.ds(g * 16, 16)] = z16
                return c2

            lax.fori_loop(0, STRIP * (CW // 16), zz, 0)
            for t in range(NSTRIP):
                pltpu.sync_copy(
                    strip_v,
                    acc_sh.at[pl.ds(ss * ROWS_PER_TILE + t * STRIP, STRIP)])
            plsc.subcore_barrier()

            def load(k, p):
                pltpu.sync_copy(ed_hbm.at[bbase + k], eb[p])
                for i in range(PC_B // 16):
                    sl = pl.ds(i * 16, 16)
                    idxd[p][sl] = eb[p][0, sl]
                    gidx[p][sl] = eb[p][1, sl] + chunk * N
                    exv[p][sl] = plsc.bitcast(eb[p][2, sl], jnp.float32)
                pltpu.async_copy(tab_hbm.at[gidx[p]], rows[p], sems[p])

            def proc(k, p):
                pltpu.make_async_copy(
                    tab_hbm.at[gidx[p]], rows[p], sems[p]).wait()
                rp, ep = rows[p], exv[p]

                def scale(eg, c2):
                    ev = ep[pl.ds(eg * 16, 16)]
                    for eu in range(16):
                        e = eg * 16 + eu
                        w = ev[eu]
                        for g in range(CW // 16):
                            rp[e, pl.ds(g * 16, 16)] = (
                                rp[e, pl.ds(g * 16, 16)] * w)
                    return c2

                lax.fori_loop(0, PC_B // 16, scale, 0)
                pltpu.sync_copy(rp, acc_sh.at[idxd[p]], add=True)

            load(0, 0)

            def pair(kk, carry):
                k0 = kk * 2
                load(k0 + 1, 1)
                proc(k0, 0)
                load(k0 + 2, 0)
                proc(k0 + 1, 1)
                return carry

            lax.fori_loop(0, (PC_NBLK - 1) // 2, pair, 0)
            proc(PC_NBLK - 1, (PC_NBLK - 1) % 2)

            plsc.subcore_barrier()
            for t in range(NSTRIP):
                r0 = ss * ROWS_PER_TILE + t * STRIP
                pltpu.sync_copy(acc_sh.at[pl.ds(r0, STRIP)], strip_v)
                pltpu.sync_copy(strip_v, out_hbm.at[pl.ds(chunk * N + r0, STRIP)])
            plsc.subcore_barrier()

    return pass_c


def _dense_prologue(x, wl, bl, wr, br):
    def body(x_ref, wl_ref, bl_ref, wr_ref, br_ref, xl_ref, xr_ref):
        xb = x_ref[...]
        dn = (((1,), (1,)), ((), ()))
        xl_ref[...] = lax.dot_general(
            xb, wl_ref[...], dn, preferred_element_type=jnp.float32) + bl_ref[...]
        xr_ref[...] = lax.dot_general(
            xb, wr_ref[...], dn, preferred_element_type=jnp.float32) + br_ref[...]

    return pl.pallas_call(
        body,
        grid=(NROW_BLK,),
        in_specs=[
            pl.BlockSpec((ROW_BLK, DIN), lambda j: (j, 0)),
            pl.BlockSpec((DH, DIN), lambda j: (0, 0)),
            pl.BlockSpec((1, DH), lambda j: (0, 0)),
            pl.BlockSpec((DH, DIN), lambda j: (0, 0)),
            pl.BlockSpec((1, DH), lambda j: (0, 0)),
        ],
        out_specs=[
            pl.BlockSpec((ROW_BLK, DH), lambda j: (j, 0)),
            pl.BlockSpec((ROW_BLK, DH), lambda j: (j, 0)),
        ],
        out_shape=[
            jax.ShapeDtypeStruct((N, DH), jnp.float32),
            jax.ShapeDtypeStruct((N, DH), jnp.float32),
        ],
    )(x, wl, bl.reshape(1, DH), wr, br.reshape(1, DH))


def _gat_finish(num, den, bias):
    def body(num_ref, den_ref, bias_ref, a0_ref, st_ref, acc):
        j = pl.program_id(0)
        a = num_ref[...] / (den_ref[...] + 1e-16) + bias_ref[...]
        a = jnp.maximum(a, 0.0)
        a0_ref[...] = a

        @pl.when(j == 0)
        def _():
            acc[...] = jnp.zeros_like(acc)

        acc[0:1, :] = acc[0:1, :] + jnp.sum(a, axis=0, keepdims=True)
        acc[1:2, :] = acc[1:2, :] + jnp.sum(a * a, axis=0, keepdims=True)

        @pl.when(j == pl.num_programs(0) - 1)
        def _():
            st_ref[...] = acc[...]

    return pl.pallas_call(
        body,
        grid=(NROW_BLK,),
        in_specs=[
            pl.BlockSpec((ROW_BLK, DH), lambda j: (j, 0)),
            pl.BlockSpec((ROW_BLK, 1), lambda j: (j, 0)),
            pl.BlockSpec((1, DH), lambda j: (0, 0)),
        ],
        out_specs=[
            pl.BlockSpec((ROW_BLK, DH), lambda j: (j, 0)),
            pl.BlockSpec((8, DH), lambda j: (0, 0)),
        ],
        out_shape=[
            jax.ShapeDtypeStruct((N, DH), jnp.float32),
            jax.ShapeDtypeStruct((8, DH), jnp.float32),
        ],
        scratch_shapes=[pltpu.VMEM((8, DH), jnp.float32)],
    )(num, den, bias.reshape(1, DH))


def _bn_matmul_relu(a, st, g, be, w, b, d_in, d_out):
    def body(a_ref, st_ref, g_ref, be_ref, w_ref, b_ref, y_ref, stout_ref, acc):
        j = pl.program_id(0)
        m = st_ref[0:1, :] / N
        v = st_ref[1:2, :] / N - m * m
        inv = lax.rsqrt(v + EPS_BN)
        xn = (a_ref[...] - m) * inv * g_ref[...] + be_ref[...]
        dn = (((1,), (1,)), ((), ()))
        y = lax.dot_general(
            xn, w_ref[...], dn, preferred_element_type=jnp.float32) + b_ref[...]
        y = jnp.maximum(y, 0.0)
        y_ref[...] = y

        @pl.when(j == 0)
        def _():
            acc[...] = jnp.zeros_like(acc)

        acc[0:1, :] = acc[0:1, :] + jnp.sum(y, axis=0, keepdims=True)
        acc[1:2, :] = acc[1:2, :] + jnp.sum(y * y, axis=0, keepdims=True)

        @pl.when(j == pl.num_programs(0) - 1)
        def _():
            stout_ref[...] = acc[...]

    return pl.pallas_call(
        body,
        grid=(NROW_BLK,),
        in_specs=[
            pl.BlockSpec((ROW_BLK, d_in), lambda j: (j, 0)),
            pl.BlockSpec((8, d_in), lambda j: (0, 0)),
            pl.BlockSpec((1, d_in), lambda j: (0, 0)),
            pl.BlockSpec((1, d_in), lambda j: (0, 0)),
            pl.BlockSpec((d_out, d_in), lambda j: (0, 0)),
            pl.BlockSpec((1, d_out), lambda j: (0, 0)),
        ],
        out_specs=[
            pl.BlockSpec((ROW_BLK, d_out), lambda j: (j, 0)),
            pl.BlockSpec((8, d_out), lambda j: (0, 0)),
        ],
        out_shape=[
            jax.ShapeDtypeStruct((N, d_out), jnp.float32),
            jax.ShapeDtypeStruct((8, d_out), jnp.float32),
        ],
        scratch_shapes=[pltpu.VMEM((8, d_out), jnp.float32)],
    )(a, st, g.reshape(1, d_in), be.reshape(1, d_in), w, b.reshape(1, d_out))


def _head(a, st, g, be, w, b, d_in, d_out):
    def body(a_ref, st_ref, g_ref, be_ref, w_ref, b_ref, sig_ref, lsm_ref):
        m = st_ref[0:1, :] / N
        v = st_ref[1:2, :] / N - m * m
        inv = lax.rsqrt(v + EPS_BN)
        xn = (a_ref[...] - m) * inv * g_ref[...] + be_ref[...]
        dn = (((1,), (1,)), ((), ()))
        z = lax.dot_general(
            xn, w_ref[...], dn, preferred_element_type=jnp.float32) + b_ref[...]
        sig_ref[...] = 1.0 / (1.0 + jnp.exp(-z))
        zm = jnp.max(z, axis=1, keepdims=True)
        lse = zm + jnp.log(jnp.sum(jnp.exp(z - zm), axis=1, keepdims=True))
        lsm_ref[...] = z - lse

    return pl.pallas_call(
        body,
        grid=(NROW_BLK,),
        in_specs=[
            pl.BlockSpec((ROW_BLK, d_in), lambda j: (j, 0)),
            pl.BlockSpec((8, d_in), lambda j: (0, 0)),
            pl.BlockSpec((1, d_in), lambda j: (0, 0)),
            pl.BlockSpec((1, d_in), lambda j: (0, 0)),
            pl.BlockSpec((d_out, d_in), lambda j: (0, 0)),
            pl.BlockSpec((1, d_out), lambda j: (0, 0)),
        ],
        out_specs=[
            pl.BlockSpec((ROW_BLK, d_out), lambda j: (j, 0)),
            pl.BlockSpec((ROW_BLK, d_out), lambda j: (j, 0)),
        ],
        out_shape=[
            jax.ShapeDtypeStruct((N, d_out), jnp.float32),
            jax.ShapeDtypeStruct((N, d_out), jnp.float32),
        ],
    )(a, st, g.reshape(1, d_in), be.reshape(1, d_in), w, b.reshape(1, d_out))


def kernel(data_0, data_1, edge_index_0, edge_index_1, batch_0, batch_1,
           train, params):
    x = data_1
    sl = jnp.arange(N, dtype=jnp.int32)
    pad = jnp.zeros((E_PAD - E_TRUE,), jnp.int32)
    src = jnp.concatenate([edge_index_1[0], sl, pad])
    dst = jnp.concatenate([edge_index_1[1], sl, pad])

    xl, xr = _dense_prologue(x, params['Wl1'], params['bl1'],
                             params['Wr1'], params['br1'])

    xl4 = xl.reshape(N, NCHUNK, DIN).transpose(1, 0, 2)
    ones = jnp.ones((NCHUNK, N, 1), jnp.float32)
    zpad = jnp.zeros((NCHUNK, N, CW - DIN - 1), jnp.float32)
    tab = jnp.concatenate([xl4, ones, zpad], axis=2).reshape(NCHUNK * N, CW)

    ed_a = jnp.stack([src.reshape(-1, PA_B), dst.reshape(-1, PA_B)], axis=1)
    ex = _build_pass_a()(ed_a, xl, xr, params['att1'])
    ed_c = jnp.stack(
        [dst.reshape(-1, PC_B), src.reshape(-1, PC_B),
         jax.lax.bitcast_convert_type(ex, jnp.int32).reshape(-1, PC_B)],
        axis=1)
    acc = _build_pass_c()(ed_c, tab)

    acc4 = acc.reshape(NCHUNK, N, CW)
    num = acc4[:, :, :DIN].transpose(1, 0, 2).reshape(N, DH)
    den = acc4[0, :, DIN:DIN + 1]

    a0, st0 = _gat_finish(num, den, params['bias1'])
    a1, st1 = _bn_matmul_relu(a0, st0, params['bng1'], params['bnb1'],
                              params['W1'], params['b1'], DH, 256)
    a2, st2 = _bn_matmul_relu(a1, st1, params['g1'], params['be1'],
                              params['W2'], params['b2'], 256, 128)
    sig, lsm = _head(a2, st2, params['g2'], params['be2'],
                     params['W3'], params['b3'], 128, 8)
    return (sig, lsm)
